# baseline (speedup 1.0000x reference)
import functools

import jax
import jax.numpy as jnp
from jax import lax
from jax.experimental import pallas as pl
from jax.experimental.pallas import tpu as pltpu
from jax.experimental.pallas import tpu_sc as plsc

_NBUF = 4


def _positional_table(seq_len, d_model):
    pos = jnp.arange(seq_len, dtype=jnp.float32)[:, None]
    i = jnp.arange(d_model // 2, dtype=jnp.float32)[None, :]
    angle = pos / jnp.power(10000.0, (2.0 * i) / d_model)
    pe = jnp.zeros((seq_len, d_model), dtype=jnp.float32)
    pe = pe.at[:, 0::2].set(jnp.sin(angle))
    pe = pe.at[:, 1::2].set(jnp.cos(angle))
    return pe


@functools.lru_cache(maxsize=None)
def _make_sc_embed(V, D, B, L):
    info = plsc.get_sparse_core_info()
    NC, NS = info.num_cores, info.num_subcores
    NW = NC * NS
    assert B % NW == 0
    b_per_w = B // NW
    C = 64
    NB = _NBUF
    assert b_per_w % (NB * C) == 0 and C % 8 == 0
    n_chunks = b_per_w // C
    n_groups = n_chunks // NB

    mesh = plsc.VectorSubcoreMesh(core_axis_name="c", subcore_axis_name="s")

    @functools.partial(
        pl.kernel,
        mesh=mesh,
        out_type=jax.ShapeDtypeStruct((B, D), jnp.float32),
        scratch_types=[
            pltpu.VMEM((b_per_w + 16,), jnp.int32),
            pltpu.VMEM((NB, C, D), jnp.float32),
            pltpu.VMEM((NB, C, D), jnp.float32),
            pltpu.VMEM((2 * L, D), jnp.float32),
            [pltpu.SemaphoreType.DMA] * NB,
            [pltpu.SemaphoreType.DMA] * NB,
        ],
    )
    def k(table_hbm, idx_hbm, pe_hbm, out_hbm,
          idx_v, gbuf, wbuf, pe_v, gsems, wsems):
        wid = lax.axis_index("s") * NC + lax.axis_index("c")
        base = wid * b_per_w
        pltpu.sync_copy(idx_hbm.at[pl.ds(base, b_per_w)], idx_v.at[pl.ds(0, b_per_w)])
        pltpu.sync_copy(pe_hbm, pe_v)

        def start_gather(cg, b):
            off = pl.multiple_of(cg * C, C)

            def row_fetch(j16, carry):
                r0 = j16 * 16
                v16 = idx_v[pl.ds(off + r0, 16)]
                for lane in range(16):
                    pltpu.async_copy(
                        table_hbm.at[v16[lane]], gbuf.at[b, r0 + lane],
                        gsems[b],
                    )
                return carry

            lax.fori_loop(0, C // 16, row_fetch, 0, unroll=C // 16)

        def wait_gather(cg, b):
            pltpu.make_async_copy(
                table_hbm.at[pl.ds(0, C)], gbuf.at[b], gsems[b]
            ).wait()

        def start_write(cg, b):
            off = pl.multiple_of(cg * C, C)
            return pltpu.async_copy(
                wbuf.at[b], out_hbm.at[pl.ds(base + off, C)], wsems[b]
            )

        def wait_write(cg, b):
            pltpu.make_async_copy(
                wbuf.at[b],
                out_hbm.at[pl.ds(base + pl.multiple_of(cg * C, C), C)],
                wsems[b],
            ).wait()

        def add_pe(b, po):
            def row_body(r, rcarry):
                for c in range(D // 16):
                    sl = pl.ds(c * 16, 16)
                    wbuf[b, r, sl] = gbuf[b, r, sl] + pe_v[po + r, sl]
                return rcarry

            lax.fori_loop(0, C, row_body, 0, unroll=4)

        for b in range(NB):
            start_gather(b, b)

        for b in range(NB):
            wait_gather(b, b)
            add_pe(b, (b * C) % L)
            start_gather(NB + b, b)
            start_write(b, b)

        def group(G, carry):
            for b in range(NB):
                cg = NB * G + b
                wait_gather(cg, b)
                wait_write(cg - NB, b)
                add_pe(b, lax.rem(cg * C, L))
                start_gather(cg + NB, b)
                start_write(cg, b)
            return carry

        lax.fori_loop(1, n_groups - 1, group, 0)

        for b in range(NB):
            cg = n_chunks - NB + b
            wait_gather(cg, b)
            wait_write(cg - NB, b)
            add_pe(b, (cg * C) % L)
            start_write(cg, b).wait()

    return k


def kernel(x, token_table):
    B, L = x.shape
    V, D = token_table.shape
    xf = x.reshape(-1).astype(jnp.int32)
    pe = _positional_table(L, D)
    pe2 = jnp.concatenate([pe, pe], axis=0)
    out = _make_sc_embed(V, D, B * L, L)(token_table, xf, pe2)
    return out.reshape(B, L, D)


# speedup vs baseline: 1.0158x; 1.0158x over previous
import functools

import jax
import jax.numpy as jnp
from jax import lax
from jax.experimental import pallas as pl
from jax.experimental.pallas import tpu as pltpu
from jax.experimental.pallas import tpu_sc as plsc

_NBUF = 4


def _positional_table(seq_len, d_model):
    pos = jnp.arange(seq_len, dtype=jnp.float32)[:, None]
    i = jnp.arange(d_model // 2, dtype=jnp.float32)[None, :]
    angle = pos / jnp.power(10000.0, (2.0 * i) / d_model)
    pe = jnp.zeros((seq_len, d_model), dtype=jnp.float32)
    pe = pe.at[:, 0::2].set(jnp.sin(angle))
    pe = pe.at[:, 1::2].set(jnp.cos(angle))
    return pe


@functools.lru_cache(maxsize=None)
def _make_sc_embed(V, D, B, L):
    info = plsc.get_sparse_core_info()
    NC, NS = info.num_cores, info.num_subcores
    NW = NC * NS
    assert B % NW == 0
    b_per_w = B // NW
    C = 64
    NB = _NBUF
    assert b_per_w % (NB * C) == 0 and C % 8 == 0
    n_chunks = b_per_w // C
    n_groups = n_chunks // NB

    mesh = plsc.VectorSubcoreMesh(core_axis_name="c", subcore_axis_name="s")

    @functools.partial(
        pl.kernel,
        mesh=mesh,
        out_type=jax.ShapeDtypeStruct((B, D), jnp.float32),
        scratch_types=[
            pltpu.VMEM((b_per_w + 16,), jnp.int32),
            pltpu.VMEM((NB, C, D), jnp.float32),
            pltpu.VMEM((NB, C, D), jnp.float32),
            pltpu.VMEM((2 * L, D), jnp.float32),
            [pltpu.SemaphoreType.DMA] * NB,
            [pltpu.SemaphoreType.DMA] * NB,
        ],
    )
    def k(table_hbm, idx_hbm, pe_hbm, out_hbm,
          idx_v, gbuf, wbuf, pe_v, gsems, wsems):
        wid = lax.axis_index("s") * NC + lax.axis_index("c")
        base = wid * b_per_w
        pltpu.sync_copy(idx_hbm.at[pl.ds(base, b_per_w)], idx_v.at[pl.ds(0, b_per_w)])
        pltpu.sync_copy(pe_hbm, pe_v)

        def start_gather(cg, b):
            off = pl.multiple_of(cg * C, C)

            def row_fetch(j16, carry):
                r0 = j16 * 16
                v16 = idx_v[pl.ds(off + r0, 16)]
                for lane in range(16):
                    pltpu.async_copy(
                        table_hbm.at[v16[lane]], gbuf.at[b, r0 + lane],
                        gsems[b],
                    )
                return carry

            lax.fori_loop(0, C // 16, row_fetch, 0)

        def wait_gather(cg, b):
            pltpu.make_async_copy(
                table_hbm.at[pl.ds(0, C)], gbuf.at[b], gsems[b]
            ).wait()

        def start_write(cg, b):
            off = pl.multiple_of(cg * C, C)
            return pltpu.async_copy(
                wbuf.at[b], out_hbm.at[pl.ds(base + off, C)], wsems[b]
            )

        def wait_write(cg, b):
            pltpu.make_async_copy(
                wbuf.at[b],
                out_hbm.at[pl.ds(base + pl.multiple_of(cg * C, C), C)],
                wsems[b],
            ).wait()

        def add_pe(b, po):
            def row_body(r, rcarry):
                for c in range(D // 16):
                    sl = pl.ds(c * 16, 16)
                    wbuf[b, r, sl] = gbuf[b, r, sl] + pe_v[po + r, sl]
                return rcarry

            lax.fori_loop(0, C, row_body, 0, unroll=4)

        for b in range(NB):
            start_gather(b, b)

        for b in range(NB):
            wait_gather(b, b)
            add_pe(b, (b * C) % L)
            start_gather(NB + b, b)
            start_write(b, b)

        def group(G, carry):
            for b in range(NB):
                cg = NB * G + b
                wait_gather(cg, b)
                wait_write(cg - NB, b)
                add_pe(b, lax.rem(cg * C, L))
                start_gather(cg + NB, b)
                start_write(cg, b)
            return carry

        lax.fori_loop(1, n_groups - 1, group, 0)

        for b in range(NB):
            cg = n_chunks - NB + b
            wait_gather(cg, b)
            wait_write(cg - NB, b)
            add_pe(b, (cg * C) % L)
            start_write(cg, b).wait()

    return k


def kernel(x, token_table):
    B, L = x.shape
    V, D = token_table.shape
    xf = x.reshape(-1).astype(jnp.int32)
    pe = _positional_table(L, D)
    pe2 = jnp.concatenate([pe, pe], axis=0)
    out = _make_sc_embed(V, D, B * L, L)(token_table, xf, pe2)
    return out.reshape(B, L, D)


# speedup vs baseline: 1.1681x; 1.1499x over previous
import functools

import jax
import jax.numpy as jnp
from jax import lax
from jax.experimental import pallas as pl
from jax.experimental.pallas import tpu as pltpu
from jax.experimental.pallas import tpu_sc as plsc

_NBUF = 4


def _positional_table(seq_len, d_model):
    pos = jnp.arange(seq_len, dtype=jnp.float32)[:, None]
    i = jnp.arange(d_model // 2, dtype=jnp.float32)[None, :]
    angle = pos / jnp.power(10000.0, (2.0 * i) / d_model)
    pe = jnp.zeros((seq_len, d_model), dtype=jnp.float32)
    pe = pe.at[:, 0::2].set(jnp.sin(angle))
    pe = pe.at[:, 1::2].set(jnp.cos(angle))
    return pe


@functools.lru_cache(maxsize=None)
def _make_sc_embed(V, D, B, L):
    info = plsc.get_sparse_core_info()
    NC, NS = info.num_cores, info.num_subcores
    NW = NC * NS
    assert B % NW == 0
    b_per_w = B // NW
    C = 64
    NB = _NBUF
    assert b_per_w % (NB * C) == 0 and C % 8 == 0
    n_chunks = b_per_w // C
    n_groups = n_chunks // NB

    mesh = plsc.VectorSubcoreMesh(core_axis_name="c", subcore_axis_name="s")

    @functools.partial(
        pl.kernel,
        mesh=mesh,
        out_type=jax.ShapeDtypeStruct((B, D), jnp.float32),
        scratch_types=[
            pltpu.VMEM((b_per_w + 16,), jnp.int32),
            pltpu.VMEM((NB, C, D), jnp.float32),
            pltpu.VMEM((NB, C, D), jnp.float32),
            pltpu.VMEM((2 * L, D), jnp.float32),
            [pltpu.SemaphoreType.DMA] * NB,
            [pltpu.SemaphoreType.DMA] * NB,
        ],
    )
    def k(table_hbm, idx_hbm, pe_hbm, out_hbm,
          idx_v, gbuf, wbuf, pe_v, gsems, wsems):
        wid = lax.axis_index("s") * NC + lax.axis_index("c")
        base = wid * b_per_w
        pltpu.sync_copy(idx_hbm.at[pl.ds(base, b_per_w)], idx_v.at[pl.ds(0, b_per_w)])
        pltpu.sync_copy(pe_hbm, pe_v)

        def start_gather(cg, b):
            off = pl.multiple_of(cg * C, C)

            def row_fetch(j16, carry):
                r0 = j16 * 16
                v16 = idx_v[pl.ds(off + r0, 16)]
                for lane in range(16):
                    vv = v16[lane]
                    blk = vv // (V // 2)
                    rr = vv - blk * (V // 2)
                    pltpu.async_copy(
                        table_hbm.at[blk, rr], gbuf.at[b, r0 + lane],
                        gsems[b],
                    )
                return carry

            lax.fori_loop(0, C // 16, row_fetch, 0)

        def wait_gather(cg, b):
            pltpu.make_async_copy(
                table_hbm.at[0, pl.ds(0, C)], gbuf.at[b], gsems[b]
            ).wait()

        def start_write(cg, b):
            off = pl.multiple_of(cg * C, C)
            return pltpu.async_copy(
                wbuf.at[b], out_hbm.at[pl.ds(base + off, C)], wsems[b]
            )

        def wait_write(cg, b):
            pltpu.make_async_copy(
                wbuf.at[b],
                out_hbm.at[pl.ds(base + pl.multiple_of(cg * C, C), C)],
                wsems[b],
            ).wait()

        def add_pe(b, po):
            def row_body(r, rcarry):
                for c in range(D // 16):
                    sl = pl.ds(c * 16, 16)
                    wbuf[b, r, sl] = gbuf[b, r, sl] + pe_v[po + r, sl]
                return rcarry

            lax.fori_loop(0, C, row_body, 0, unroll=4)

        for b in range(NB):
            start_gather(b, b)

        for b in range(NB):
            wait_gather(b, b)
            add_pe(b, (b * C) % L)
            start_gather(NB + b, b)
            start_write(b, b)

        def group(G, carry):
            for b in range(NB):
                cg = NB * G + b
                wait_gather(cg, b)
                wait_write(cg - NB, b)
                add_pe(b, lax.rem(cg * C, L))
                start_gather(cg + NB, b)
                start_write(cg, b)
            return carry

        lax.fori_loop(1, n_groups - 1, group, 0)

        for b in range(NB):
            cg = n_chunks - NB + b
            wait_gather(cg, b)
            wait_write(cg - NB, b)
            add_pe(b, (cg * C) % L)
            start_write(cg, b).wait()

    return k


def kernel(x, token_table):
    B, L = x.shape
    V, D = token_table.shape
    xf = x.reshape(-1).astype(jnp.int32)
    pe = _positional_table(L, D)
    pe2 = jnp.concatenate([pe, pe], axis=0)
    table3 = token_table.reshape(2, V // 2, D)
    out = _make_sc_embed(V, D, B * L, L)(table3, xf, pe2)
    return out.reshape(B, L, D)
